# baseline (speedup 1.0000x reference)
import jax
import jax.numpy as jnp
from jax import lax
from jax.experimental import pallas as pl
from jax.experimental.pallas import tpu as pltpu
from jax.experimental.pallas import tpu_sc as plsc

N_ROWS = 100001
H = 128
BLK = 2048
NBLK = 49
NPAD = NBLK * BLK
NW = 32
E = 4096 * 200
E_PER_W = E // NW
CHUNK = 12800
NCHUNK = E_PER_W // CHUNK
MASK_PER_W = NPAD // NW
NLOC = 4096


def _sc_hist_body(l_hbm, loc_hbm, zeros_hbm, partials_hbm, maskcnt_hbm,
                  counts_v, idx_v, mask_v, loc_v):
    wid = lax.axis_index("s") * 2 + lax.axis_index("c")

    pltpu.sync_copy(zeros_hbm, counts_v)

    base_e = wid * E_PER_W
    for c in range(NCHUNK):
        pltpu.sync_copy(l_hbm.at[pl.ds(base_e + c * CHUNK, CHUNK)], idx_v)

        def hist(j, c_):
            idx = idx_v[pl.ds(j * 16, 16)]
            cnt, last = plsc.scan_count(idx)
            plsc.addupdate_scatter(counts_v, [idx], cnt.astype(jnp.float32),
                                   mask=last)
            return c_
        lax.fori_loop(0, CHUNK // 16, hist, 0, unroll=8)
    pltpu.sync_copy(counts_v, partials_hbm.at[pl.ds(wid * NPAD, NPAD)])

    pltpu.sync_copy(zeros_hbm.at[pl.ds(0, MASK_PER_W)], mask_v)
    pltpu.sync_copy(loc_hbm, loc_v)
    mbase = wid * MASK_PER_W

    def mloop(j, c_):
        idx = loc_v[pl.ds(j * 16, 16)]
        cnt, last = plsc.scan_count(idx)
        inr = (idx >= mbase) & (idx < mbase + MASK_PER_W)
        plsc.addupdate_scatter(mask_v, [idx - mbase], cnt.astype(jnp.float32),
                               mask=last & inr)
        return c_
    lax.fori_loop(0, NLOC // 16, mloop, 0, unroll=4)
    pltpu.sync_copy(mask_v, maskcnt_hbm.at[pl.ds(mbase, MASK_PER_W)])


def _sc_hist(l_flat, loc_idx, zeros):
    mesh = plsc.VectorSubcoreMesh(core_axis_name="c", subcore_axis_name="s")
    return pl.kernel(
        _sc_hist_body,
        out_type=(jax.ShapeDtypeStruct((NW * NPAD,), jnp.float32),
                  jax.ShapeDtypeStruct((NPAD,), jnp.float32)),
        mesh=mesh,
        scratch_types=[
            pltpu.VMEM((NPAD,), jnp.float32),
            pltpu.VMEM((CHUNK,), jnp.int32),
            pltpu.VMEM((MASK_PER_W,), jnp.float32),
            pltpu.VMEM((NLOC,), jnp.int32),
        ],
        compiler_params=pltpu.CompilerParams(needs_layout_passes=False),
    )(l_flat, loc_idx, zeros)


_DN_ROWMAT = (((1,), (0,)), ((), ()))
_DN_OUTER = (((0,), (0,)), ((), ()))


def _tc_body(part_ref, p_ref, w1_ref, b1_ref, w2_ref, b2_ref, w3_ref, b3_ref,
             m_ref, l_ref, out_ref, lnew_ref, acc_ref):
    p = pl.program_id(0)
    j = pl.program_id(1)

    @pl.when(p == 0)
    def _():
        @pl.when(j == 0)
        def _():
            acc_ref[...] = jnp.zeros_like(acc_ref)

        w = jnp.sum(part_ref[...], axis=0, keepdims=True)
        rows = j * BLK + lax.broadcasted_iota(jnp.int32, (BLK, H), 0)
        pb = jnp.where(rows < N_ROWS, p_ref[...], 0.0)
        acc_ref[...] += lax.dot_general(w, pb, _DN_ROWMAT,
                                        preferred_element_type=jnp.float32)

        @pl.when(j == NBLK - 1)
        def _():
            e = acc_ref[...]
            x = lax.dot_general(e, w1_ref[...], _DN_ROWMAT,
                                preferred_element_type=jnp.float32) + b1_ref[...]
            x = jnp.maximum(x, 0.0)
            x = lax.dot_general(x, w2_ref[...], _DN_ROWMAT,
                                preferred_element_type=jnp.float32) + b2_ref[...]
            x = jnp.maximum(x, 0.0)
            z = lax.dot_general(x, w3_ref[...], _DN_ROWMAT,
                                preferred_element_type=jnp.float32) + b3_ref[...]
            out_ref[...] = 1.0 / (1.0 + jnp.exp(-z))

    @pl.when(p == 1)
    def _():
        m = (m_ref[0] > 0.0).astype(jnp.float32)
        e2d = lax.dot_general(m, acc_ref[...], _DN_OUTER,
                              preferred_element_type=jnp.float32)
        m2d = lax.dot_general(m, jnp.ones((1, H), jnp.float32), _DN_OUTER,
                              preferred_element_type=jnp.float32)
        lnew_ref[...] = l_ref[...] * (1.0 - m2d) + e2d


def _tc_fused(partials, p_embed, w1, b1, w2, b2, w3, b3, mask3d, l_embed):
    full = lambda s: pl.BlockSpec(s, lambda p, j: (0, 0))
    return pl.pallas_call(
        _tc_body,
        grid=(2, NBLK),
        in_specs=[
            pl.BlockSpec((NW, BLK), lambda p, j: (0, j * (1 - p))),
            pl.BlockSpec((BLK, H), lambda p, j: (j * (1 - p), 0)),
            full((H, H // 2)), full((1, H // 2)),
            full((H // 2, H // 4)), full((1, H // 4)),
            full((H // 4, 1)), full((1, 1)),
            pl.BlockSpec((1, 1, BLK), lambda p, j: (j * p, 0, 0)),
            pl.BlockSpec((BLK, H), lambda p, j: (j * p, 0)),
        ],
        out_specs=[full((1, 1)),
                   pl.BlockSpec((BLK, H), lambda p, j: (j * p, 0))],
        out_shape=[jax.ShapeDtypeStruct((1, 1), jnp.float32),
                   jax.ShapeDtypeStruct((N_ROWS, H), jnp.float32)],
        scratch_shapes=[pltpu.VMEM((1, H), jnp.float32)],
        compiler_params=pltpu.CompilerParams(
            dimension_semantics=("arbitrary", "arbitrary")),
    )(partials, p_embed, w1, b1, w2, b2, w3, b3, mask3d, l_embed)


def kernel(locIndexes, l_input, P_embed, L_embed, W1, b1, W2, b2, W3, b3):
    loc = locIndexes.astype(jnp.int32)
    l_flat = l_input.reshape(-1).astype(jnp.int32)
    zeros = jnp.zeros((NPAD,), jnp.float32)
    partials_flat, maskcnt = _sc_hist(l_flat, loc, zeros)
    out11, l_new = _tc_fused(partials_flat.reshape(NW, NPAD), P_embed,
                             W1, b1.reshape(1, -1), W2, b2.reshape(1, -1),
                             W3, b3.reshape(1, 1),
                             maskcnt.reshape(NBLK, 1, BLK), L_embed)
    return (out11.reshape(()), l_new)


# speedup vs baseline: 1.2619x; 1.2619x over previous
import jax
import jax.numpy as jnp
from jax import lax
from jax.experimental import pallas as pl
from jax.experimental.pallas import tpu as pltpu
from jax.experimental.pallas import tpu_sc as plsc

N_ROWS = 100001
H = 128
BLK = 4096
NBLK = 25
NPAD = NBLK * BLK
NW = 32
E = 4096 * 200
E_PER_W = E // NW
CHUNK = 12800
NCHUNK = E_PER_W // CHUNK
MASK_PER_W = NPAD // NW
NLOC = 4096


def _sc_hist_body(l_hbm, loc_hbm, partials_hbm, maskcnt_hbm,
                  counts_v, idx_v, mask_v, loc_v, sem):
    wid = lax.axis_index("s") * 2 + lax.axis_index("c")
    zeros16 = jnp.zeros((16,), jnp.float32)

    base_e = wid * E_PER_W
    cp0 = pltpu.make_async_copy(l_hbm.at[pl.ds(base_e, CHUNK)], idx_v, sem)
    cp0.start()

    def zero_counts(i, c):
        counts_v[pl.ds(i * 16, 16)] = zeros16
        return c
    lax.fori_loop(0, NPAD // 16, zero_counts, 0, unroll=8)

    def zero_mask(i, c):
        mask_v[pl.ds(i * 16, 16)] = zeros16
        return c
    lax.fori_loop(0, MASK_PER_W // 16, zero_mask, 0, unroll=8)
    cp0.wait()

    for c in range(NCHUNK):
        if c > 0:
            pltpu.sync_copy(l_hbm.at[pl.ds(base_e + c * CHUNK, CHUNK)], idx_v)

        def hist(j, c_):
            idx = idx_v[pl.ds(j * 16, 16)]
            cnt, last = plsc.scan_count(idx)
            plsc.addupdate_scatter(counts_v, [idx], cnt.astype(jnp.float32),
                                   mask=last)
            return c_
        lax.fori_loop(0, CHUNK // 16, hist, 0, unroll=8)
    pltpu.sync_copy(counts_v, partials_hbm.at[pl.ds(wid * NPAD, NPAD)])

    pltpu.sync_copy(loc_hbm, loc_v)
    mbase = wid * MASK_PER_W

    def mloop(j, c_):
        idx = loc_v[pl.ds(j * 16, 16)]
        cnt, last = plsc.scan_count(idx)
        inr = (idx >= mbase) & (idx < mbase + MASK_PER_W)
        plsc.addupdate_scatter(mask_v, [idx - mbase], cnt.astype(jnp.float32),
                               mask=last & inr)
        return c_
    lax.fori_loop(0, NLOC // 16, mloop, 0, unroll=4)
    pltpu.sync_copy(mask_v, maskcnt_hbm.at[pl.ds(mbase, MASK_PER_W)])


def _sc_hist(l_flat, loc_idx):
    mesh = plsc.VectorSubcoreMesh(core_axis_name="c", subcore_axis_name="s")
    return pl.kernel(
        _sc_hist_body,
        out_type=(jax.ShapeDtypeStruct((NW * NPAD,), jnp.float32),
                  jax.ShapeDtypeStruct((NPAD,), jnp.float32)),
        mesh=mesh,
        scratch_types=[
            pltpu.VMEM((NPAD,), jnp.float32),
            pltpu.VMEM((CHUNK,), jnp.int32),
            pltpu.VMEM((MASK_PER_W,), jnp.float32),
            pltpu.VMEM((NLOC,), jnp.int32),
            pltpu.SemaphoreType.DMA,
        ],
        compiler_params=pltpu.CompilerParams(needs_layout_passes=False),
    )(l_flat, loc_idx)


_DN_ROWMAT = (((1,), (0,)), ((), ()))
_DN_OUTER = (((0,), (0,)), ((), ()))


def _tc_body(part_ref, p_ref, w1_ref, b1_ref, w2_ref, b2_ref, w3_ref, b3_ref,
             m_ref, l_ref, out_ref, lnew_ref, acc_ref):
    p = pl.program_id(0)
    j = pl.program_id(1)

    @pl.when(p == 0)
    def _():
        @pl.when(j == 0)
        def _():
            acc_ref[...] = jnp.zeros_like(acc_ref)

        w = jnp.sum(part_ref[...], axis=0, keepdims=True)
        rows = j * BLK + lax.broadcasted_iota(jnp.int32, (BLK, H), 0)
        pb = jnp.where(rows < N_ROWS, p_ref[...], 0.0)
        acc_ref[...] += lax.dot_general(w, pb, _DN_ROWMAT,
                                        preferred_element_type=jnp.float32)

        @pl.when(j == NBLK - 1)
        def _():
            e = acc_ref[...]
            x = lax.dot_general(e, w1_ref[...], _DN_ROWMAT,
                                preferred_element_type=jnp.float32) + b1_ref[...]
            x = jnp.maximum(x, 0.0)
            x = lax.dot_general(x, w2_ref[...], _DN_ROWMAT,
                                preferred_element_type=jnp.float32) + b2_ref[...]
            x = jnp.maximum(x, 0.0)
            z = lax.dot_general(x, w3_ref[...], _DN_ROWMAT,
                                preferred_element_type=jnp.float32) + b3_ref[...]
            out_ref[...] = 1.0 / (1.0 + jnp.exp(-z))

    @pl.when(p == 1)
    def _():
        m = (m_ref[0] > 0.0).astype(jnp.float32)
        e2d = lax.dot_general(m, acc_ref[...], _DN_OUTER,
                              preferred_element_type=jnp.float32)
        m2d = lax.dot_general(m, jnp.ones((1, H), jnp.float32), _DN_OUTER,
                              preferred_element_type=jnp.float32)
        lnew_ref[...] = jnp.where(m2d > 0.5, e2d, l_ref[...])


def _tc_fused(partials, p_embed, w1, b1, w2, b2, w3, b3, mask3d, l_embed):
    full = lambda s: pl.BlockSpec(s, lambda p, j: (0, 0))
    return pl.pallas_call(
        _tc_body,
        grid=(2, NBLK),
        in_specs=[
            pl.BlockSpec((NW, BLK),
                         lambda p, j: (0, j * (1 - p) + (NBLK - 1) * p)),
            pl.BlockSpec((BLK, H),
                         lambda p, j: (j * (1 - p) + (NBLK - 1) * p, 0)),
            full((H, H // 2)), full((1, H // 2)),
            full((H // 2, H // 4)), full((1, H // 4)),
            full((H // 4, 1)), full((1, 1)),
            pl.BlockSpec((1, 1, BLK), lambda p, j: (j * p, 0, 0)),
            pl.BlockSpec((BLK, H), lambda p, j: (j * p, 0)),
        ],
        out_specs=[full((1, 1)),
                   pl.BlockSpec((BLK, H), lambda p, j: (j * p, 0))],
        out_shape=[jax.ShapeDtypeStruct((1, 1), jnp.float32),
                   jax.ShapeDtypeStruct((N_ROWS, H), jnp.float32)],
        scratch_shapes=[pltpu.VMEM((1, H), jnp.float32)],
        compiler_params=pltpu.CompilerParams(
            dimension_semantics=("arbitrary", "arbitrary")),
    )(partials, p_embed, w1, b1, w2, b2, w3, b3, mask3d, l_embed)


def kernel(locIndexes, l_input, P_embed, L_embed, W1, b1, W2, b2, W3, b3):
    loc = locIndexes.astype(jnp.int32)
    l_flat = l_input.reshape(-1).astype(jnp.int32)
    partials_flat, maskcnt = _sc_hist(l_flat, loc)
    out11, l_new = _tc_fused(partials_flat.reshape(NW, NPAD), P_embed,
                             W1, b1.reshape(1, -1), W2, b2.reshape(1, -1),
                             W3, b3.reshape(1, 1),
                             maskcnt.reshape(NBLK, 1, BLK), L_embed)
    return (out11.reshape(()), l_new)


# speedup vs baseline: 1.3678x; 1.0839x over previous
import jax
import jax.numpy as jnp
from jax import lax
from jax.experimental import pallas as pl
from jax.experimental.pallas import tpu as pltpu
from jax.experimental.pallas import tpu_sc as plsc

N_ROWS = 100001
H = 128
BLK = 4096
NBLK = 25
NPAD = NBLK * BLK
NW = 32
E = 4096 * 200
E_PER_W = E // NW
CHUNK = 12800
NCHUNK = E_PER_W // CHUNK
MASK_PER_W = NPAD // NW
NLOC = 4096


def _sc_hist_body(l_hbm, loc_hbm, partials_hbm, maskcnt_hbm,
                  counts_v, idx_v, mask_v, loc_v, sem):
    wid = lax.axis_index("s") * 2 + lax.axis_index("c")
    zeros16 = jnp.zeros((16,), jnp.float32)

    base_e = wid * E_PER_W
    cp0 = pltpu.make_async_copy(l_hbm.at[pl.ds(base_e, CHUNK)], idx_v, sem)
    cp0.start()

    def zero_counts(i, c):
        counts_v[pl.ds(i * 16, 16)] = zeros16
        return c
    lax.fori_loop(0, NPAD // 16, zero_counts, 0, unroll=8)

    def zero_mask(i, c):
        mask_v[pl.ds(i * 16, 16)] = zeros16
        return c
    lax.fori_loop(0, BLK // 16, zero_mask, 0, unroll=8)
    cp0.wait()

    for c in range(NCHUNK):
        if c > 0:
            pltpu.sync_copy(l_hbm.at[pl.ds(base_e + c * CHUNK, CHUNK)], idx_v)

        def hist(j, c_):
            idx = idx_v[pl.ds(j * 16, 16)]
            cnt, last = plsc.scan_count(idx)
            plsc.addupdate_scatter(counts_v, [idx], cnt.astype(jnp.float32),
                                   mask=last)
            return c_
        lax.fori_loop(0, CHUNK // 16, hist, 0, unroll=8)
    pltpu.sync_copy(counts_v, partials_hbm.at[wid])

    @pl.when(wid < NBLK)
    def _():
        pltpu.sync_copy(loc_hbm, loc_v)
        mbase = wid * BLK

        def mloop(j, c_):
            idx = loc_v[pl.ds(j * 16, 16)]
            cnt, last = plsc.scan_count(idx)
            inr = (idx >= mbase) & (idx < mbase + BLK)
            plsc.addupdate_scatter(mask_v, [idx - mbase],
                                   cnt.astype(jnp.float32), mask=last & inr)
            return c_
        lax.fori_loop(0, NLOC // 16, mloop, 0, unroll=4)
        pltpu.sync_copy(mask_v, maskcnt_hbm.at[wid, 0])


def _sc_hist(l_flat, loc_idx):
    mesh = plsc.VectorSubcoreMesh(core_axis_name="c", subcore_axis_name="s")
    return pl.kernel(
        _sc_hist_body,
        out_type=(jax.ShapeDtypeStruct((NW, NPAD), jnp.float32),
                  jax.ShapeDtypeStruct((NBLK, 1, BLK), jnp.float32)),
        mesh=mesh,
        scratch_types=[
            pltpu.VMEM((NPAD,), jnp.float32),
            pltpu.VMEM((CHUNK,), jnp.int32),
            pltpu.VMEM((BLK,), jnp.float32),
            pltpu.VMEM((NLOC,), jnp.int32),
            pltpu.SemaphoreType.DMA,
        ],
        compiler_params=pltpu.CompilerParams(needs_layout_passes=False),
    )(l_flat, loc_idx)


_DN_ROWMAT = (((1,), (0,)), ((), ()))
_DN_OUTER = (((0,), (0,)), ((), ()))


def _tc_body(part_ref, p_ref, w1_ref, b1_ref, w2_ref, b2_ref, w3_ref, b3_ref,
             m_ref, l_ref, out_ref, lnew_ref, acc_ref):
    p = pl.program_id(0)
    j = pl.program_id(1)

    @pl.when(p == 0)
    def _():
        @pl.when(j == 0)
        def _():
            acc_ref[...] = jnp.zeros_like(acc_ref)

        w = jnp.sum(part_ref[...], axis=0, keepdims=True)
        rows = j * BLK + lax.broadcasted_iota(jnp.int32, (BLK, H), 0)
        pb = jnp.where(rows < N_ROWS, p_ref[...], 0.0)
        acc_ref[...] += lax.dot_general(w, pb, _DN_ROWMAT,
                                        preferred_element_type=jnp.float32)

        @pl.when(j == NBLK - 1)
        def _():
            e = acc_ref[...]
            x = lax.dot_general(e, w1_ref[...], _DN_ROWMAT,
                                preferred_element_type=jnp.float32) + b1_ref[...]
            x = jnp.maximum(x, 0.0)
            x = lax.dot_general(x, w2_ref[...], _DN_ROWMAT,
                                preferred_element_type=jnp.float32) + b2_ref[...]
            x = jnp.maximum(x, 0.0)
            z = lax.dot_general(x, w3_ref[...], _DN_ROWMAT,
                                preferred_element_type=jnp.float32) + b3_ref[...]
            out_ref[...] = 1.0 / (1.0 + jnp.exp(-z))

    @pl.when(p == 1)
    def _():
        m = (m_ref[0] > 0.0).astype(jnp.float32)
        e2d = lax.dot_general(m, acc_ref[...], _DN_OUTER,
                              preferred_element_type=jnp.float32)
        m2d = lax.dot_general(m, jnp.ones((1, H), jnp.float32), _DN_OUTER,
                              preferred_element_type=jnp.float32)
        lnew_ref[...] = jnp.where(m2d > 0.5, e2d, l_ref[...])


def _tc_fused(partials, p_embed, w1, b1, w2, b2, w3, b3, mask3d, l_embed):
    full = lambda s: pl.BlockSpec(s, lambda p, j: (0, 0))
    return pl.pallas_call(
        _tc_body,
        grid=(2, NBLK),
        in_specs=[
            pl.BlockSpec((NW, BLK),
                         lambda p, j: (0, j * (1 - p) + (NBLK - 1) * p)),
            pl.BlockSpec((BLK, H),
                         lambda p, j: (j * (1 - p) + (NBLK - 1) * p, 0)),
            full((H, H // 2)), full((1, H // 2)),
            full((H // 2, H // 4)), full((1, H // 4)),
            full((H // 4, 1)), full((1, 1)),
            pl.BlockSpec((1, 1, BLK), lambda p, j: (j * p, 0, 0)),
            pl.BlockSpec((BLK, H), lambda p, j: (j * p, 0)),
        ],
        out_specs=[full((1, 1)),
                   pl.BlockSpec((BLK, H), lambda p, j: (j * p, 0))],
        out_shape=[jax.ShapeDtypeStruct((1, 1), jnp.float32),
                   jax.ShapeDtypeStruct((N_ROWS, H), jnp.float32)],
        scratch_shapes=[pltpu.VMEM((1, H), jnp.float32)],
        compiler_params=pltpu.CompilerParams(
            dimension_semantics=("arbitrary", "arbitrary")),
    )(partials, p_embed, w1, b1, w2, b2, w3, b3, mask3d, l_embed)


def kernel(locIndexes, l_input, P_embed, L_embed, W1, b1, W2, b2, W3, b3):
    loc = locIndexes.astype(jnp.int32)
    l_flat = l_input.reshape(-1).astype(jnp.int32)
    partials, maskcnt = _sc_hist(l_flat, loc)
    out11, l_new = _tc_fused(partials, P_embed,
                             W1, b1.reshape(1, -1), W2, b2.reshape(1, -1),
                             W3, b3.reshape(1, 1), maskcnt, L_embed)
    return (out11.reshape(()), l_new)


# speedup vs baseline: 1.7708x; 1.2946x over previous
import jax
import jax.numpy as jnp
from jax import lax
from jax.experimental import pallas as pl
from jax.experimental.pallas import tpu as pltpu
from jax.experimental.pallas import tpu_sc as plsc

N_ROWS = 100001
H = 128
BLK = 4096
NBLK = 25
NPAD = NBLK * BLK
NW = 32
E = 4096 * 200
E_PER_W = E // NW
CHUNK = 12800
NCHUNK = E_PER_W // CHUNK
NLOC = 4096
LOC_PER_W = NLOC // NW


def _sc_hist_body(l_hbm, partials_hbm, counts_v, idx_v, sem):
    wid = lax.axis_index("s") * 2 + lax.axis_index("c")
    zeros16 = jnp.zeros((16,), jnp.float32)

    base_e = wid * E_PER_W
    cp0 = pltpu.make_async_copy(l_hbm.at[pl.ds(base_e, CHUNK)], idx_v, sem)
    cp0.start()

    def zero_counts(i, c):
        counts_v[pl.ds(i * 16, 16)] = zeros16
        return c
    lax.fori_loop(0, NPAD // 16, zero_counts, 0, unroll=8)
    cp0.wait()

    for c in range(NCHUNK):
        if c > 0:
            pltpu.sync_copy(l_hbm.at[pl.ds(base_e + c * CHUNK, CHUNK)], idx_v)

        def hist(j, c_):
            idx = idx_v[pl.ds(j * 16, 16)]
            cnt, last = plsc.scan_count(idx)
            plsc.addupdate_scatter(counts_v, [idx], cnt.astype(jnp.float32),
                                   mask=last)
            return c_
        lax.fori_loop(0, CHUNK // 16, hist, 0, unroll=8)
    pltpu.sync_copy(counts_v, partials_hbm.at[wid])


def _sc_hist(l_flat):
    mesh = plsc.VectorSubcoreMesh(core_axis_name="c", subcore_axis_name="s")
    return pl.kernel(
        _sc_hist_body,
        out_type=jax.ShapeDtypeStruct((NW, NPAD), jnp.float32),
        mesh=mesh,
        scratch_types=[
            pltpu.VMEM((NPAD,), jnp.float32),
            pltpu.VMEM((CHUNK,), jnp.int32),
            pltpu.SemaphoreType.DMA,
        ],
        compiler_params=pltpu.CompilerParams(needs_layout_passes=False),
    )(l_flat)


def _sc_scatter_body(loc_hbm, emb_hbm, tbl_ref, loc_v, emb_v, rows_v, sem):
    wid = lax.axis_index("s") * 2 + lax.axis_index("c")
    base = wid * LOC_PER_W
    cp0 = pltpu.make_async_copy(loc_hbm.at[pl.ds(base, LOC_PER_W)], loc_v, sem)
    cp0.start()
    pltpu.sync_copy(emb_hbm, emb_v)

    def rep(i, c):
        for k in range(H // 16):
            rows_v[i, pl.ds(k * 16, 16)] = emb_v[pl.ds(k * 16, 16)]
        return c
    lax.fori_loop(0, LOC_PER_W, rep, 0, unroll=2)
    cp0.wait()
    pltpu.async_copy(rows_v, tbl_ref.at[loc_v], sem).wait()


def _sc_scatter(loc_idx, emb_flat, tbl_state):
    mesh = plsc.VectorSubcoreMesh(core_axis_name="c", subcore_axis_name="s")
    return pl.kernel(
        _sc_scatter_body,
        out_type=(),
        mesh=mesh,
        scratch_types=[
            pltpu.VMEM((LOC_PER_W,), jnp.int32),
            pltpu.VMEM((H,), jnp.float32),
            pltpu.VMEM((LOC_PER_W, H), jnp.float32),
            pltpu.SemaphoreType.DMA,
        ],
        compiler_params=pltpu.CompilerParams(needs_layout_passes=False),
    )(loc_idx, emb_flat, tbl_state)


_DN_ROWMAT = (((1,), (0,)), ((), ()))


def _copy_body(l_ref, out_ref):
    out_ref[...] = l_ref[...]


def _tc_copy(l_embed):
    return pl.pallas_call(
        _copy_body,
        grid=(NBLK,),
        in_specs=[pl.BlockSpec((BLK, H), lambda j: (j, 0))],
        out_specs=pl.BlockSpec((BLK, H), lambda j: (j, 0)),
        out_shape=jax.ShapeDtypeStruct((N_ROWS, H), jnp.float32),
        compiler_params=pltpu.CompilerParams(
            dimension_semantics=("parallel",)),
    )(l_embed)


def _sweep_body(part_ref, p_ref, w1_ref, b1_ref, w2_ref, b2_ref, w3_ref,
                b3_ref, out_ref, emb_ref, acc_ref):
    j = pl.program_id(0)

    @pl.when(j == 0)
    def _():
        acc_ref[...] = jnp.zeros_like(acc_ref)

    w = jnp.sum(part_ref[...], axis=0, keepdims=True)
    rows = j * BLK + lax.broadcasted_iota(jnp.int32, (BLK, H), 0)
    pb = jnp.where(rows < N_ROWS, p_ref[...], 0.0)
    acc_ref[...] += lax.dot_general(w, pb, _DN_ROWMAT,
                                    preferred_element_type=jnp.float32)

    @pl.when(j == NBLK - 1)
    def _():
        e = acc_ref[...]
        emb_ref[...] = e
        x = lax.dot_general(e, w1_ref[...], _DN_ROWMAT,
                            preferred_element_type=jnp.float32) + b1_ref[...]
        x = jnp.maximum(x, 0.0)
        x = lax.dot_general(x, w2_ref[...], _DN_ROWMAT,
                            preferred_element_type=jnp.float32) + b2_ref[...]
        x = jnp.maximum(x, 0.0)
        z = lax.dot_general(x, w3_ref[...], _DN_ROWMAT,
                            preferred_element_type=jnp.float32) + b3_ref[...]
        out_ref[...] = 1.0 / (1.0 + jnp.exp(-z))


def _tc_sweep(partials, p_embed, w1, b1, w2, b2, w3, b3):
    full = lambda s: pl.BlockSpec(s, lambda j: (0, 0))
    return pl.pallas_call(
        _sweep_body,
        grid=(NBLK,),
        in_specs=[
            pl.BlockSpec((NW, BLK), lambda j: (0, j)),
            pl.BlockSpec((BLK, H), lambda j: (j, 0)),
            full((H, H // 2)), full((1, H // 2)),
            full((H // 2, H // 4)), full((1, H // 4)),
            full((H // 4, 1)), full((1, 1)),
        ],
        out_specs=[full((1, 1)), full((1, H))],
        out_shape=[jax.ShapeDtypeStruct((1, 1), jnp.float32),
                   jax.ShapeDtypeStruct((1, H), jnp.float32)],
        scratch_shapes=[pltpu.VMEM((1, H), jnp.float32)],
        compiler_params=pltpu.CompilerParams(
            dimension_semantics=("arbitrary",)),
    )(partials, p_embed, w1, b1, w2, b2, w3, b3)


def kernel(locIndexes, l_input, P_embed, L_embed, W1, b1, W2, b2, W3, b3):
    loc = locIndexes.astype(jnp.int32)
    l_flat = l_input.reshape(-1).astype(jnp.int32)
    lcopy = _tc_copy(L_embed)
    partials = _sc_hist(l_flat)
    out11, emb = _tc_sweep(partials, P_embed,
                           W1, b1.reshape(1, -1), W2, b2.reshape(1, -1),
                           W3, b3.reshape(1, 1))
    tbl = jax.new_ref(lcopy)
    _sc_scatter(loc, emb.reshape(H), tbl)
    l_new = jax.freeze(tbl)
    return (out11.reshape(()), l_new)


# speedup vs baseline: 1.8700x; 1.0560x over previous
import jax
import jax.numpy as jnp
from jax import lax
from jax.experimental import pallas as pl
from jax.experimental.pallas import tpu as pltpu
from jax.experimental.pallas import tpu_sc as plsc

N_ROWS = 100001
H = 128
BLK = 4096
NBLK = 25
NPAD = NBLK * BLK
NW = 32
NSEQ = 4096
SEQ = 200
ROWS_PER_W = NSEQ // NW
NLOC = 4096
LOC_PER_W = NLOC // NW


def _sc_hist_body(l_hbm, partials_hbm, counts_v, idx_v, sem):
    wid = lax.axis_index("s") * 2 + lax.axis_index("c")
    zeros16 = jnp.zeros((16,), jnp.float32)
    tail_ok = lax.broadcasted_iota(jnp.int32, (16,), 0) >= 8

    half = ROWS_PER_W // 2
    cp0 = pltpu.make_async_copy(
        l_hbm.at[pl.ds(wid * ROWS_PER_W, half)], idx_v, sem)
    cp0.start()

    def zero_counts(i, c):
        counts_v[pl.ds(i * 16, 16)] = zeros16
        return c
    lax.fori_loop(0, NPAD // 16, zero_counts, 0, unroll=8)
    cp0.wait()

    def hist(r, c_):
        for k in range(12):
            idx = idx_v[r, pl.ds(k * 16, 16)]
            cnt, last = plsc.scan_count(idx)
            plsc.addupdate_scatter(counts_v, [idx], cnt.astype(jnp.float32),
                                   mask=last)
        idx = idx_v[r, pl.ds(SEQ - 16, 16)]
        cnt, last = plsc.scan_count(idx, tail_ok)
        plsc.addupdate_scatter(counts_v, [idx], cnt.astype(jnp.float32),
                               mask=last & tail_ok)
        return c_

    for c in range(2):
        if c > 0:
            pltpu.sync_copy(
                l_hbm.at[pl.ds(wid * ROWS_PER_W + half, half)], idx_v)
        lax.fori_loop(0, half, hist, 0, unroll=2)
    pltpu.sync_copy(counts_v, partials_hbm.at[wid])


def _sc_hist(l_input):
    mesh = plsc.VectorSubcoreMesh(core_axis_name="c", subcore_axis_name="s")
    return pl.kernel(
        _sc_hist_body,
        out_type=jax.ShapeDtypeStruct((NW, NPAD), jnp.float32),
        mesh=mesh,
        scratch_types=[
            pltpu.VMEM((NPAD,), jnp.float32),
            pltpu.VMEM((ROWS_PER_W // 2, SEQ), jnp.int32),
            pltpu.SemaphoreType.DMA,
        ],
        compiler_params=pltpu.CompilerParams(needs_layout_passes=False),
    )(l_input)


def _sc_scatter_body(loc_hbm, emb_hbm, tbl_ref, loc_v, emb_v, rows_v, sem):
    wid = lax.axis_index("s") * 2 + lax.axis_index("c")
    base = wid * LOC_PER_W
    cp0 = pltpu.make_async_copy(loc_hbm.at[pl.ds(base, LOC_PER_W)], loc_v, sem)
    cp0.start()
    pltpu.sync_copy(emb_hbm, emb_v)

    def rep(i, c):
        for k in range(H // 16):
            rows_v[i, pl.ds(k * 16, 16)] = emb_v[pl.ds(k * 16, 16)]
        return c
    lax.fori_loop(0, LOC_PER_W, rep, 0, unroll=2)
    cp0.wait()
    pltpu.async_copy(rows_v, tbl_ref.at[loc_v], sem).wait()


def _sc_scatter(loc_idx, emb_flat, tbl_state):
    mesh = plsc.VectorSubcoreMesh(core_axis_name="c", subcore_axis_name="s")
    return pl.kernel(
        _sc_scatter_body,
        out_type=(),
        mesh=mesh,
        scratch_types=[
            pltpu.VMEM((LOC_PER_W,), jnp.int32),
            pltpu.VMEM((H,), jnp.float32),
            pltpu.VMEM((LOC_PER_W, H), jnp.float32),
            pltpu.SemaphoreType.DMA,
        ],
        compiler_params=pltpu.CompilerParams(needs_layout_passes=False),
    )(loc_idx, emb_flat, tbl_state)


_DN_ROWMAT = (((1,), (0,)), ((), ()))


def _copy_body(l_ref, out_ref):
    out_ref[...] = l_ref[...]


def _tc_copy(l_embed):
    return pl.pallas_call(
        _copy_body,
        grid=(NBLK,),
        in_specs=[pl.BlockSpec((BLK, H), lambda j: (j, 0))],
        out_specs=pl.BlockSpec((BLK, H), lambda j: (j, 0)),
        out_shape=jax.ShapeDtypeStruct((N_ROWS, H), jnp.float32),
        compiler_params=pltpu.CompilerParams(
            dimension_semantics=("parallel",)),
    )(l_embed)


def _sweep_body(part_ref, p_ref, w1_ref, b1_ref, w2_ref, b2_ref, w3_ref,
                b3_ref, out_ref, emb_ref, acc_ref):
    j = pl.program_id(0)

    @pl.when(j == 0)
    def _():
        acc_ref[...] = jnp.zeros_like(acc_ref)

    w = jnp.sum(part_ref[...], axis=0, keepdims=True)
    rows = j * BLK + lax.broadcasted_iota(jnp.int32, (BLK, H), 0)
    pb = jnp.where(rows < N_ROWS, p_ref[...], 0.0)
    acc_ref[...] += lax.dot_general(w, pb, _DN_ROWMAT,
                                    preferred_element_type=jnp.float32)

    @pl.when(j == NBLK - 1)
    def _():
        e = acc_ref[...]
        emb_ref[...] = e
        x = lax.dot_general(e, w1_ref[...], _DN_ROWMAT,
                            preferred_element_type=jnp.float32) + b1_ref[...]
        x = jnp.maximum(x, 0.0)
        x = lax.dot_general(x, w2_ref[...], _DN_ROWMAT,
                            preferred_element_type=jnp.float32) + b2_ref[...]
        x = jnp.maximum(x, 0.0)
        z = lax.dot_general(x, w3_ref[...], _DN_ROWMAT,
                            preferred_element_type=jnp.float32) + b3_ref[...]
        out_ref[...] = 1.0 / (1.0 + jnp.exp(-z))


def _tc_sweep(partials, p_embed, w1, b1, w2, b2, w3, b3):
    full = lambda s: pl.BlockSpec(s, lambda j: (0, 0))
    return pl.pallas_call(
        _sweep_body,
        grid=(NBLK,),
        in_specs=[
            pl.BlockSpec((NW, BLK), lambda j: (0, j)),
            pl.BlockSpec((BLK, H), lambda j: (j, 0)),
            full((H, H // 2)), full((1, H // 2)),
            full((H // 2, H // 4)), full((1, H // 4)),
            full((H // 4, 1)), full((1, 1)),
        ],
        out_specs=[full((1, 1)), full((1, H))],
        out_shape=[jax.ShapeDtypeStruct((1, 1), jnp.float32),
                   jax.ShapeDtypeStruct((1, H), jnp.float32)],
        scratch_shapes=[pltpu.VMEM((1, H), jnp.float32)],
        compiler_params=pltpu.CompilerParams(
            dimension_semantics=("arbitrary",)),
    )(partials, p_embed, w1, b1, w2, b2, w3, b3)


def kernel(locIndexes, l_input, P_embed, L_embed, W1, b1, W2, b2, W3, b3):
    loc = locIndexes.astype(jnp.int32)
    lcopy = _tc_copy(L_embed)
    partials = _sc_hist(l_input.astype(jnp.int32))
    out11, emb = _tc_sweep(partials, P_embed,
                           W1, b1.reshape(1, -1), W2, b2.reshape(1, -1),
                           W3, b3.reshape(1, 1))
    tbl = jax.new_ref(lcopy)
    _sc_scatter(loc, emb.reshape(H), tbl)
    l_new = jax.freeze(tbl)
    return (out11.reshape(()), l_new)


# speedup vs baseline: 2.0554x; 1.0991x over previous
import jax
import jax.numpy as jnp
from jax import lax
from jax.experimental import pallas as pl
from jax.experimental.pallas import tpu as pltpu
from jax.experimental.pallas import tpu_sc as plsc

N_ROWS = 100001
H = 128
BLK = 8192
NBLK = 13
NPAD = NBLK * BLK
NW = 32
NSEQ = 4096
SEQ = 200
ROWS_PER_W = NSEQ // NW
NLOC = 4096
LOC_PER_W = NLOC // NW


def _sc_hist_body(l_hbm, partials_hbm, counts_v, idx_v, sem):
    wid = lax.axis_index("s") * 2 + lax.axis_index("c")
    zeros16 = jnp.zeros((16,), jnp.float32)
    tail_ok = lax.broadcasted_iota(jnp.int32, (16,), 0) >= 8

    half = ROWS_PER_W // 2
    cp0 = pltpu.make_async_copy(
        l_hbm.at[pl.ds(wid * ROWS_PER_W, half)], idx_v, sem)
    cp0.start()

    def zero_counts(i, c):
        counts_v[pl.ds(i * 16, 16)] = zeros16
        return c
    lax.fori_loop(0, NPAD // 16, zero_counts, 0, unroll=8)
    cp0.wait()

    def hist(r, c_):
        for k in range(12):
            idx = idx_v[r, pl.ds(k * 16, 16)]
            cnt, last = plsc.scan_count(idx)
            plsc.addupdate_scatter(counts_v, [idx], cnt.astype(jnp.float32),
                                   mask=last)
        idx = idx_v[r, pl.ds(SEQ - 16, 16)]
        cnt, last = plsc.scan_count(idx, tail_ok)
        plsc.addupdate_scatter(counts_v, [idx], cnt.astype(jnp.float32),
                               mask=last & tail_ok)
        return c_

    for c in range(2):
        if c > 0:
            pltpu.sync_copy(
                l_hbm.at[pl.ds(wid * ROWS_PER_W + half, half)], idx_v)
        lax.fori_loop(0, half, hist, 0, unroll=2)
    pltpu.sync_copy(counts_v, partials_hbm.at[wid])


def _sc_hist(l_input):
    mesh = plsc.VectorSubcoreMesh(core_axis_name="c", subcore_axis_name="s")
    return pl.kernel(
        _sc_hist_body,
        out_type=jax.ShapeDtypeStruct((NW, NPAD), jnp.float32),
        mesh=mesh,
        scratch_types=[
            pltpu.VMEM((NPAD,), jnp.float32),
            pltpu.VMEM((ROWS_PER_W // 2, SEQ), jnp.int32),
            pltpu.SemaphoreType.DMA,
        ],
        compiler_params=pltpu.CompilerParams(needs_layout_passes=False),
    )(l_input)


def _sc_scatter_body(loc_hbm, emb_hbm, tbl_ref, loc_v, emb_v, rows_v, sem):
    wid = lax.axis_index("s") * 2 + lax.axis_index("c")
    base = wid * LOC_PER_W
    cp0 = pltpu.make_async_copy(loc_hbm.at[pl.ds(base, LOC_PER_W)], loc_v, sem)
    cp0.start()
    pltpu.sync_copy(emb_hbm, emb_v)

    def rep(i, c):
        for k in range(H // 16):
            rows_v[i, pl.ds(k * 16, 16)] = emb_v[pl.ds(k * 16, 16)]
        return c
    lax.fori_loop(0, LOC_PER_W, rep, 0, unroll=2)
    cp0.wait()
    pltpu.async_copy(rows_v, tbl_ref.at[loc_v], sem).wait()


def _sc_scatter(loc_idx, emb_flat, tbl_state):
    mesh = plsc.VectorSubcoreMesh(core_axis_name="c", subcore_axis_name="s")
    return pl.kernel(
        _sc_scatter_body,
        out_type=(),
        mesh=mesh,
        scratch_types=[
            pltpu.VMEM((LOC_PER_W,), jnp.int32),
            pltpu.VMEM((H,), jnp.float32),
            pltpu.VMEM((LOC_PER_W, H), jnp.float32),
            pltpu.SemaphoreType.DMA,
        ],
        compiler_params=pltpu.CompilerParams(needs_layout_passes=False),
    )(loc_idx, emb_flat, tbl_state)


_DN_ROWMAT = (((1,), (0,)), ((), ()))


def _copy_body(l_ref, out_ref):
    out_ref[...] = l_ref[...]


def _tc_copy(l_embed):
    return pl.pallas_call(
        _copy_body,
        grid=(NBLK,),
        in_specs=[pl.BlockSpec((BLK, H), lambda j: (j, 0))],
        out_specs=pl.BlockSpec((BLK, H), lambda j: (j, 0)),
        out_shape=jax.ShapeDtypeStruct((N_ROWS, H), jnp.float32),
        compiler_params=pltpu.CompilerParams(
            dimension_semantics=("parallel",)),
    )(l_embed)


def _sweep_body(part_ref, p_ref, w1_ref, b1_ref, w2_ref, b2_ref, w3_ref,
                b3_ref, out_ref, emb_ref, acc_ref):
    j = pl.program_id(0)

    @pl.when(j == 0)
    def _():
        acc_ref[...] = jnp.zeros_like(acc_ref)

    w = jnp.sum(part_ref[...], axis=0, keepdims=True)
    rows = j * BLK + lax.broadcasted_iota(jnp.int32, (BLK, H), 0)
    pb = jnp.where(rows < N_ROWS, p_ref[...], 0.0)
    acc_ref[...] += lax.dot_general(w, pb, _DN_ROWMAT,
                                    preferred_element_type=jnp.float32)

    @pl.when(j == NBLK - 1)
    def _():
        e = acc_ref[...]
        emb_ref[...] = e
        x = lax.dot_general(e, w1_ref[...], _DN_ROWMAT,
                            preferred_element_type=jnp.float32) + b1_ref[...]
        x = jnp.maximum(x, 0.0)
        x = lax.dot_general(x, w2_ref[...], _DN_ROWMAT,
                            preferred_element_type=jnp.float32) + b2_ref[...]
        x = jnp.maximum(x, 0.0)
        z = lax.dot_general(x, w3_ref[...], _DN_ROWMAT,
                            preferred_element_type=jnp.float32) + b3_ref[...]
        out_ref[...] = 1.0 / (1.0 + jnp.exp(-z))


def _tc_sweep(partials, p_embed, w1, b1, w2, b2, w3, b3):
    full = lambda s: pl.BlockSpec(s, lambda j: (0, 0))
    return pl.pallas_call(
        _sweep_body,
        grid=(NBLK,),
        in_specs=[
            pl.BlockSpec((NW, BLK), lambda j: (0, j)),
            pl.BlockSpec((BLK, H), lambda j: (j, 0)),
            full((H, H // 2)), full((1, H // 2)),
            full((H // 2, H // 4)), full((1, H // 4)),
            full((H // 4, 1)), full((1, 1)),
        ],
        out_specs=[full((1, 1)), full((1, H))],
        out_shape=[jax.ShapeDtypeStruct((1, 1), jnp.float32),
                   jax.ShapeDtypeStruct((1, H), jnp.float32)],
        scratch_shapes=[pltpu.VMEM((1, H), jnp.float32)],
        compiler_params=pltpu.CompilerParams(
            dimension_semantics=("arbitrary",)),
    )(partials, p_embed, w1, b1, w2, b2, w3, b3)


def kernel(locIndexes, l_input, P_embed, L_embed, W1, b1, W2, b2, W3, b3):
    loc = locIndexes.astype(jnp.int32)
    lcopy = _tc_copy(L_embed)
    partials = _sc_hist(l_input.astype(jnp.int32))
    out11, emb = _tc_sweep(partials, P_embed,
                           W1, b1.reshape(1, -1), W2, b2.reshape(1, -1),
                           W3, b3.reshape(1, 1))
    tbl = jax.new_ref(lcopy)
    _sc_scatter(loc, emb.reshape(H), tbl)
    l_new = jax.freeze(tbl)
    return (out11.reshape(()), l_new)
